# baseline (speedup 1.0000x reference)
import functools

import jax
import jax.numpy as jnp
from jax import lax
from jax.experimental import pallas as pl
from jax.experimental.pallas import tpu as pltpu
from jax.experimental.pallas import tpu_sc as plsc

N = 10000
DEG = 32
D = 128
OUT = 128
BN = 400
BN2 = 1000

N_SC = 4000
R = 8
NBLK = N_SC // R
NW = 32
BASE_BLKS = NBLK // NW
EXTRA = NBLK - BASE_BLKS * NW


def _make_sc_agg_body(n_rows, row_off):
    nblk_total = n_rows // R
    base_blks = nblk_total // NW
    extra = nblk_total - base_blks * NW

    def _sc_agg_body(neigh_hbm, agg_hbm, buf0, buf1, out0, out1,
                     sin0, sin1, sout0, sout1):
        w = lax.axis_index("s") * 2 + lax.axis_index("c")
        nblk = base_blks + jnp.where(w < extra, 1, 0)
        base = w * base_blks + jnp.minimum(w, extra)

        def start_in(blk, buf, sem):
            pltpu.async_copy(
                neigh_hbm.at[pl.ds(row_off + blk * R, R)], buf, sem)

        def wait_in(buf, sem):
            pltpu.make_async_copy(
                neigh_hbm.at[pl.ds(0, R)], buf, sem).wait()

        def start_out(blk, buf, sem):
            pltpu.async_copy(buf, agg_hbm.at[pl.ds(blk * R, R)], sem)

        def wait_out(buf, sem):
            pltpu.make_async_copy(buf, agg_hbm.at[pl.ds(0, R)], sem).wait()

        start_in(base, buf0, sin0)

        @pl.when(nblk > 1)
        def _():
            start_in(base + 1, buf1, sin1)

        npair = (base_blks + 2) // 2

        def do_block(b, buf, outb, s_in, s_out):
            valid = b < nblk

            @pl.when(valid)
            def _():
                wait_in(buf, s_in)

                def row_body(r, _):
                    accs = [buf[r, 0, pl.ds(16 * c, 16)] for c in range(8)]
                    for k in range(1, DEG):
                        for c in range(8):
                            accs[c] = accs[c] + buf[r, k, pl.ds(16 * c, 16)]
                    for c in range(8):
                        outb[r, pl.ds(16 * c, 16)] = accs[c]
                    return 0

                lax.fori_loop(0, R, row_body, 0, unroll=2)

                @pl.when(b >= 2)
                def _():
                    wait_out(outb, s_out)

                start_out(base + b, outb, s_out)

                @pl.when(b + 2 < nblk)
                def _():
                    start_in(base + b + 2, buf, s_in)

        def pair_body(p, _):
            do_block(2 * p, buf0, out0, sin0, sout0)
            do_block(2 * p + 1, buf1, out1, sin1, sout1)
            return 0

        lax.fori_loop(0, npair, pair_body, 0)

        @pl.when(nblk >= 1)
        def _():
            wait_out(out0, sout0)

        @pl.when(nblk >= 2)
        def _():
            wait_out(out1, sout1)

    return _sc_agg_body


def _sc_aggregate(neigh_feats, n_rows=N_SC, row_off=0):
    mesh = plsc.VectorSubcoreMesh(core_axis_name="c", subcore_axis_name="s")
    f = functools.partial(
        pl.kernel,
        out_type=jax.ShapeDtypeStruct((n_rows, D), jnp.float32),
        mesh=mesh,
        scratch_types=[
            pltpu.VMEM((R, DEG, D), jnp.float32),
            pltpu.VMEM((R, DEG, D), jnp.float32),
            pltpu.VMEM((R, D), jnp.float32),
            pltpu.VMEM((R, D), jnp.float32),
            pltpu.SemaphoreType.DMA,
            pltpu.SemaphoreType.DMA,
            pltpu.SemaphoreType.DMA,
            pltpu.SemaphoreType.DMA,
        ],
    )(_make_sc_agg_body(n_rows, row_off))
    return f(neigh_feats)


def _pass1a_body(neigh_ref, self_ref, nn_ref, w_self_ref, b_self_ref,
                 w_neigh_ref, b_neigh_ref, h_ref, s1_ref, s2_ref):
    neigh = neigh_ref[...]
    agg = jnp.sum(neigh, axis=1)
    nn = nn_ref[...]
    nn = jnp.where(nn == 0.0, 1.0, nn)
    agg = agg / nn
    self_h = jnp.dot(self_ref[...], w_self_ref[...],
                     preferred_element_type=jnp.float32) + b_self_ref[...]
    agg_h = jnp.dot(agg, w_neigh_ref[...],
                    preferred_element_type=jnp.float32) + b_neigh_ref[...]
    h = jnp.concatenate([self_h, agg_h], axis=1)
    h_ref[...] = h
    ps1 = jnp.sum(h, axis=0, keepdims=True)
    ps2 = jnp.sum(h * h, axis=0, keepdims=True)

    @pl.when(pl.program_id(0) == 0)
    def _init():
        s1_ref[...] = ps1
        s2_ref[...] = ps2

    @pl.when(pl.program_id(0) != 0)
    def _acc():
        s1_ref[...] += ps1
        s2_ref[...] += ps2


def _pass1b_body(h_alias_ref, agg_ref, self_ref, nn_ref, w_self_ref,
                 b_self_ref, w_neigh_ref, b_neigh_ref, s1h_ref, s2h_ref,
                 h_ref, s1_ref, s2_ref):
    del h_alias_ref
    nn = nn_ref[...]
    nn = jnp.where(nn == 0.0, 1.0, nn)
    agg = agg_ref[...] / nn
    self_h = jnp.dot(self_ref[...], w_self_ref[...],
                     preferred_element_type=jnp.float32) + b_self_ref[...]
    agg_h = jnp.dot(agg, w_neigh_ref[...],
                    preferred_element_type=jnp.float32) + b_neigh_ref[...]
    h = jnp.concatenate([self_h, agg_h], axis=1)
    h_ref[...] = h
    ps1 = jnp.sum(h, axis=0, keepdims=True)
    ps2 = jnp.sum(h * h, axis=0, keepdims=True)

    @pl.when(pl.program_id(0) == 0)
    def _init():
        s1_ref[...] = s1h_ref[...] + ps1
        s2_ref[...] = s2h_ref[...] + ps2

    @pl.when(pl.program_id(0) != 0)
    def _acc():
        s1_ref[...] += ps1
        s2_ref[...] += ps2


def _pass1b_call(h_prev, agg_sc, self_feats, nn2, W_self, b_self2,
                 W_neigh, b_neigh2, s1_prev, s2_prev, n_rows, row_off):
    off_blk = row_off // BN
    grid_b = n_rows // BN
    return pl.pallas_call(
        _pass1b_body,
        grid=(grid_b,),
        in_specs=[
            pl.BlockSpec(memory_space=pl.ANY),
            pl.BlockSpec((BN, D), lambda i: (i, 0)),
            pl.BlockSpec((BN, D), lambda i: (i + off_blk, 0)),
            pl.BlockSpec((BN, 1), lambda i: (i + off_blk, 0)),
            pl.BlockSpec((D, OUT), lambda i: (0, 0)),
            pl.BlockSpec((1, OUT), lambda i: (0, 0)),
            pl.BlockSpec((D, OUT), lambda i: (0, 0)),
            pl.BlockSpec((1, OUT), lambda i: (0, 0)),
            pl.BlockSpec((1, 2 * OUT), lambda i: (0, 0)),
            pl.BlockSpec((1, 2 * OUT), lambda i: (0, 0)),
        ],
        out_specs=[
            pl.BlockSpec((BN, 2 * OUT), lambda i: (i + off_blk, 0)),
            pl.BlockSpec((1, 2 * OUT), lambda i: (0, 0)),
            pl.BlockSpec((1, 2 * OUT), lambda i: (0, 0)),
        ],
        out_shape=[
            jax.ShapeDtypeStruct((N, 2 * OUT), jnp.float32),
            jax.ShapeDtypeStruct((1, 2 * OUT), jnp.float32),
            jax.ShapeDtypeStruct((1, 2 * OUT), jnp.float32),
        ],
        input_output_aliases={0: 0},
    )(h_prev, agg_sc, self_feats, nn2, W_self, b_self2, W_neigh, b_neigh2,
      s1_prev, s2_prev)


def _pass2_body(h_ref, s1_ref, s2_ref, gamma_ref, beta_ref, out_ref):
    mean = s1_ref[...] / N
    var = s2_ref[...] / N - mean * mean
    scale = gamma_ref[...] * jax.lax.rsqrt(var + 1e-3)
    shift = beta_ref[...] - mean * scale
    out_ref[...] = jnp.maximum(h_ref[...] * scale + shift, 0.0)


def kernel(self_feats, neigh_feats, self_nneigh, neigh_nneigh,
           W_self, b_self, W_neigh, b_neigh, gamma, beta):
    nn2 = self_nneigh.reshape(N, 1)
    b_self2 = b_self.reshape(1, OUT)
    b_neigh2 = b_neigh.reshape(1, OUT)
    gamma2 = gamma.reshape(1, 2 * OUT)
    beta2 = beta.reshape(1, 2 * OUT)

    HALF = N_SC // 2
    agg_sc0 = _sc_aggregate(neigh_feats, HALF, 0)

    OFF = N_SC // BN
    grid_a = (N - N_SC) // BN
    h0, s1h, s2h = pl.pallas_call(
        _pass1a_body,
        grid=(grid_a,),
        in_specs=[
            pl.BlockSpec((BN, DEG, D), lambda i: (i + OFF, 0, 0)),
            pl.BlockSpec((BN, D), lambda i: (i + OFF, 0)),
            pl.BlockSpec((BN, 1), lambda i: (i + OFF, 0)),
            pl.BlockSpec((D, OUT), lambda i: (0, 0)),
            pl.BlockSpec((1, OUT), lambda i: (0, 0)),
            pl.BlockSpec((D, OUT), lambda i: (0, 0)),
            pl.BlockSpec((1, OUT), lambda i: (0, 0)),
        ],
        out_specs=[
            pl.BlockSpec((BN, 2 * OUT), lambda i: (i + OFF, 0)),
            pl.BlockSpec((1, 2 * OUT), lambda i: (0, 0)),
            pl.BlockSpec((1, 2 * OUT), lambda i: (0, 0)),
        ],
        out_shape=[
            jax.ShapeDtypeStruct((N, 2 * OUT), jnp.float32),
            jax.ShapeDtypeStruct((1, 2 * OUT), jnp.float32),
            jax.ShapeDtypeStruct((1, 2 * OUT), jnp.float32),
        ],
    )(neigh_feats, self_feats, nn2, W_self, b_self2, W_neigh, b_neigh2)

    agg_sc1 = _sc_aggregate(neigh_feats, HALF, HALF)

    h1, s1a, s2a = _pass1b_call(h0, agg_sc0, self_feats, nn2, W_self,
                                b_self2, W_neigh, b_neigh2, s1h, s2h,
                                HALF, 0)
    h, s1, s2 = _pass1b_call(h1, agg_sc1, self_feats, nn2, W_self,
                             b_self2, W_neigh, b_neigh2, s1a, s2a,
                             HALF, HALF)

    out = pl.pallas_call(
        _pass2_body,
        grid=(N // BN2,),
        in_specs=[
            pl.BlockSpec((BN2, 2 * OUT), lambda i: (i, 0)),
            pl.BlockSpec((1, 2 * OUT), lambda i: (0, 0)),
            pl.BlockSpec((1, 2 * OUT), lambda i: (0, 0)),
            pl.BlockSpec((1, 2 * OUT), lambda i: (0, 0)),
            pl.BlockSpec((1, 2 * OUT), lambda i: (0, 0)),
        ],
        out_specs=pl.BlockSpec((BN2, 2 * OUT), lambda i: (i, 0)),
        out_shape=jax.ShapeDtypeStruct((N, 2 * OUT), jnp.float32),
    )(h, s1, s2, gamma2, beta2)
    return out


# speedup vs baseline: 1.4587x; 1.4587x over previous
import jax
import jax.numpy as jnp
from jax.experimental import pallas as pl
from jax.experimental.pallas import tpu as pltpu

N = 10000
DEG = 32
D = 128
OUT = 128
BN = 400
BN2 = 1000


def _pass1_body(neigh_ref, self_ref, nn_ref, w_self_ref, b_self_ref,
                w_neigh_ref, b_neigh_ref, h_ref, s1_ref, s2_ref):
    neigh = neigh_ref[...]
    agg = jnp.sum(neigh, axis=1)
    nn = nn_ref[...]
    nn = jnp.where(nn == 0.0, 1.0, nn)
    agg = agg / nn
    self_h = jnp.dot(self_ref[...], w_self_ref[...],
                     preferred_element_type=jnp.float32) + b_self_ref[...]
    agg_h = jnp.dot(agg, w_neigh_ref[...],
                    preferred_element_type=jnp.float32) + b_neigh_ref[...]
    h = jnp.concatenate([self_h, agg_h], axis=1)
    h_ref[...] = h.astype(jnp.bfloat16)
    ps1 = jnp.sum(h, axis=0, keepdims=True)
    ps2 = jnp.sum(h * h, axis=0, keepdims=True)

    @pl.when(pl.program_id(0) == 0)
    def _init():
        s1_ref[...] = ps1
        s2_ref[...] = ps2

    @pl.when(pl.program_id(0) != 0)
    def _acc():
        s1_ref[...] += ps1
        s2_ref[...] += ps2


def _pass2_body(h_ref, s1_ref, s2_ref, gamma_ref, beta_ref, out_ref):
    mean = s1_ref[...] / N
    var = s2_ref[...] / N - mean * mean
    scale = gamma_ref[...] * jax.lax.rsqrt(var + 1e-3)
    shift = beta_ref[...] - mean * scale
    h = h_ref[...].astype(jnp.float32)
    out_ref[...] = jnp.maximum(h * scale + shift, 0.0)


def kernel(self_feats, neigh_feats, self_nneigh, neigh_nneigh,
           W_self, b_self, W_neigh, b_neigh, gamma, beta):
    nn2 = self_nneigh.reshape(N, 1)
    b_self2 = b_self.reshape(1, OUT)
    b_neigh2 = b_neigh.reshape(1, OUT)
    gamma2 = gamma.reshape(1, 2 * OUT)
    beta2 = beta.reshape(1, 2 * OUT)

    grid = N // BN
    h, s1, s2 = pl.pallas_call(
        _pass1_body,
        grid=(grid,),
        in_specs=[
            pl.BlockSpec((BN, DEG, D), lambda i: (i, 0, 0)),
            pl.BlockSpec((BN, D), lambda i: (i, 0)),
            pl.BlockSpec((BN, 1), lambda i: (i, 0)),
            pl.BlockSpec((D, OUT), lambda i: (0, 0)),
            pl.BlockSpec((1, OUT), lambda i: (0, 0)),
            pl.BlockSpec((D, OUT), lambda i: (0, 0)),
            pl.BlockSpec((1, OUT), lambda i: (0, 0)),
        ],
        out_specs=[
            pl.BlockSpec((BN, 2 * OUT), lambda i: (i, 0)),
            pl.BlockSpec((1, 2 * OUT), lambda i: (0, 0)),
            pl.BlockSpec((1, 2 * OUT), lambda i: (0, 0)),
        ],
        out_shape=[
            jax.ShapeDtypeStruct((N, 2 * OUT), jnp.bfloat16),
            jax.ShapeDtypeStruct((1, 2 * OUT), jnp.float32),
            jax.ShapeDtypeStruct((1, 2 * OUT), jnp.float32),
        ],
    )(neigh_feats, self_feats, nn2, W_self, b_self2, W_neigh, b_neigh2)

    out = pl.pallas_call(
        _pass2_body,
        grid=(N // BN2,),
        in_specs=[
            pl.BlockSpec((BN2, 2 * OUT), lambda i: (i, 0)),
            pl.BlockSpec((1, 2 * OUT), lambda i: (0, 0)),
            pl.BlockSpec((1, 2 * OUT), lambda i: (0, 0)),
            pl.BlockSpec((1, 2 * OUT), lambda i: (0, 0)),
            pl.BlockSpec((1, 2 * OUT), lambda i: (0, 0)),
        ],
        out_specs=pl.BlockSpec((BN2, 2 * OUT), lambda i: (i, 0)),
        out_shape=jax.ShapeDtypeStruct((N, 2 * OUT), jnp.float32),
    )(h, s1, s2, gamma2, beta2)
    return out
